# baseline (speedup 1.0000x reference)
import functools

import jax
import jax.numpy as jnp
from jax import lax
from jax.experimental import pallas as pl
from jax.experimental.pallas import tpu as pltpu
from jax.experimental.pallas import tpu_sc as plsc

DIGIT_DIM = 16
POS_DIM = 8
P = 81
F = 32
NW = 32
NB = 16
TBL = P * 10
XPC = NB * P


@functools.lru_cache(maxsize=None)
def _make_encoder(B: int):
    assert B % (NW * NB) == 0
    rows_per_w = B // NW
    n_chunks = rows_per_w // NB
    x_per_w = rows_per_w * P

    mesh = plsc.VectorSubcoreMesh(core_axis_name="c", subcore_axis_name="s")

    @functools.partial(
        pl.kernel,
        mesh=mesh,
        out_type=(
            jax.ShapeDtypeStruct((B * P, F), jnp.float32),
            jax.ShapeDtypeStruct((TBL, F), jnp.float32),
        ),
        compiler_params=pltpu.CompilerParams(use_tc_tiling_on_sc=False),
        scratch_types=[
            pltpu.VMEM((160,), jnp.float32),
            pltpu.VMEM((80,), jnp.float32),
            pltpu.VMEM((80,), jnp.float32),
            pltpu.VMEM((9 * 16,), jnp.float32),
            pltpu.VMEM((P * 16,), jnp.float32),
            pltpu.VMEM((TBL, F), jnp.float32),
            pltpu.VMEM((XPC,), jnp.int32),
            pltpu.VMEM((XPC,), jnp.int32),
            pltpu.VMEM((XPC,), jnp.int32),
            pltpu.VMEM((2, XPC, F), jnp.float32),
            pltpu.SemaphoreType.DMA,
            pltpu.SemaphoreType.DMA,
            pltpu.SemaphoreType.DMA,
            pltpu.SemaphoreType.DMA,
            pltpu.SemaphoreType.DMA,
        ],
    )
    def enc(x_hbm, digit_hbm, row_hbm, col_hbm, out_hbm, tbl_hbm,
            digit_v, row_v, col_v, colhi_v, pos_v, tbl_v, x0_v, x1_v,
            pat_v, rows_v, xsem0, xsem1, gsem, osem0, osem1):
        sid = lax.axis_index("s")
        wid = sid * 2 + lax.axis_index("c")
        lane = lax.iota(jnp.int32, 16)

        def x_slice(chunk):
            return x_hbm.at[pl.ds(wid * x_per_w + chunk * XPC, XPC)]

        def out_slice(chunk):
            return out_hbm.at[pl.ds(wid * x_per_w + chunk * XPC, XPC)]

        pltpu.async_copy(x_slice(0), x0_v.at[pl.ds(0, XPC)], xsem0)
        pltpu.async_copy(x_slice(1), x1_v.at[pl.ds(0, XPC)], xsem1)

        def patg(g, pv):
            pat_v[pl.ds(g * 16, 16)] = pv * 10
            pv = pv + 16
            return jnp.where(pv >= P, pv - P, pv)
        lax.fori_loop(0, XPC // 16, patg, lane)

        @pl.when(sid == 0)
        def _build_table():
            pltpu.sync_copy(digit_hbm, digit_v.at[pl.ds(0, 160)])
            pltpu.sync_copy(row_hbm, row_v.at[pl.ds(0, 72)])
            pltpu.sync_copy(col_hbm, col_v.at[pl.ds(0, 72)])
            for c in range(9):
                cv = col_v[pl.ds(c * 8, 16)]
                acc = cv * 0.0
                for i in range(8):
                    acc = jnp.where(lane == 8 + i, cv[i], acc)
                colhi_v[pl.ds(c * 16, 16)] = acc
            for p in range(P):
                r, c = p // 9, p % 9
                a = row_v[pl.ds(r * 8, 16)]
                b = colhi_v[pl.ds(c * 16, 16)]
                pos_v[pl.ds(p * 16, 16)] = jnp.where(lane < 8, a, b)
            def tb(p, carry):
                pv = pos_v[pl.ds(p * 16, 16)]
                for d in range(10):
                    tbl_v[p * 10 + d, pl.ds(0, 16)] = \
                        digit_v[pl.ds(d * 16, 16)]
                    tbl_v[p * 10 + d, pl.ds(16, 16)] = pv
                return carry
            lax.fori_loop(0, P, tb, 0)
            pltpu.sync_copy(tbl_v, tbl_hbm)
        plsc.subcore_barrier()

        def do_chunk(k, x_v, xsem, osem, chunk):
            @pl.when(chunk >= 2)
            def _drain_out():
                pltpu.make_async_copy(rows_v.at[k], out_slice(chunk), osem
                                      ).wait()
            pltpu.make_async_copy(x_slice(chunk), x_v.at[pl.ds(0, XPC)], xsem
                                  ).wait()

            @plsc.parallel_loop(0, XPC // 16, unroll=4)
            def cgrp(g):
                x_v[pl.ds(g * 16, 16)] = (
                    x_v[pl.ds(g * 16, 16)] + pat_v[pl.ds(g * 16, 16)])

            pltpu.async_copy(tbl_hbm.at[x_v], rows_v.at[k], gsem).wait()

            pltpu.async_copy(rows_v.at[k], out_slice(chunk), osem)

            @pl.when(chunk + 2 < n_chunks)
            def _prefetch_x():
                pltpu.async_copy(x_slice(chunk + 2), x_v.at[pl.ds(0, XPC)],
                                 xsem)

        def chunk_pair(ci, carry):
            do_chunk(0, x0_v, xsem0, osem0, ci * 2)
            do_chunk(1, x1_v, xsem1, osem1, ci * 2 + 1)
            return carry
        lax.fori_loop(0, n_chunks // 2, chunk_pair, 0)

        pltpu.make_async_copy(rows_v.at[0], out_slice(n_chunks - 2), osem0
                              ).wait()
        pltpu.make_async_copy(rows_v.at[1], out_slice(n_chunks - 1), osem1
                              ).wait()

    return enc


def kernel(x, digit_emb, row_emb, col_emb):
    B, p = x.shape
    assert p == P
    xf = x.reshape(-1).astype(jnp.int32)
    out, _ = _make_encoder(B)(xf, digit_emb.reshape(-1),
                              row_emb.reshape(-1), col_emb.reshape(-1))
    return out.reshape(B, P, F)


# speedup vs baseline: 1.0580x; 1.0580x over previous
import functools

import jax
import jax.numpy as jnp
from jax import lax
from jax.experimental import pallas as pl
from jax.experimental.pallas import tpu as pltpu
from jax.experimental.pallas import tpu_sc as plsc

DIGIT_DIM = 16
POS_DIM = 8
P = 81
F = 32
NW = 32
NB = 16
TBL = P * 10
XPC = NB * P


@functools.lru_cache(maxsize=None)
def _make_encoder(B: int):
    assert B % (NW * NB) == 0
    rows_per_w = B // NW
    n_chunks = rows_per_w // NB
    x_per_w = rows_per_w * P

    mesh = plsc.VectorSubcoreMesh(core_axis_name="c", subcore_axis_name="s")

    @functools.partial(
        pl.kernel,
        mesh=mesh,
        out_type=jax.ShapeDtypeStruct((B * P, F), jnp.float32),
        compiler_params=pltpu.CompilerParams(use_tc_tiling_on_sc=False),
        scratch_types=[
            pltpu.VMEM((160,), jnp.float32),
            pltpu.VMEM((80,), jnp.float32),
            pltpu.VMEM((80,), jnp.float32),
            pltpu.VMEM((9 * 16,), jnp.float32),
            pltpu.VMEM((P * 16,), jnp.float32),
            pltpu.VMEM((TBL, F), jnp.float32),
            pltpu.VMEM_SHARED((TBL, F), jnp.float32),
            pltpu.VMEM((XPC,), jnp.int32),
            pltpu.VMEM((XPC,), jnp.int32),
            pltpu.VMEM((XPC,), jnp.int32),
            pltpu.VMEM((2, XPC, F), jnp.float32),
            pltpu.SemaphoreType.DMA,
            pltpu.SemaphoreType.DMA,
            pltpu.SemaphoreType.DMA,
            pltpu.SemaphoreType.DMA,
            pltpu.SemaphoreType.DMA,
        ],
    )
    def enc(x_hbm, digit_hbm, row_hbm, col_hbm, out_hbm,
            digit_v, row_v, col_v, colhi_v, pos_v, tbl_v, stbl_v, x0_v, x1_v,
            pat_v, rows_v, xsem0, xsem1, gsem, osem0, osem1):
        sid = lax.axis_index("s")
        wid = sid * 2 + lax.axis_index("c")
        lane = lax.iota(jnp.int32, 16)

        def x_slice(chunk):
            return x_hbm.at[pl.ds(wid * x_per_w + chunk * XPC, XPC)]

        def out_slice(chunk):
            return out_hbm.at[pl.ds(wid * x_per_w + chunk * XPC, XPC)]

        pltpu.async_copy(x_slice(0), x0_v.at[pl.ds(0, XPC)], xsem0)
        pltpu.async_copy(x_slice(1), x1_v.at[pl.ds(0, XPC)], xsem1)

        def patg(g, pv):
            pat_v[pl.ds(g * 16, 16)] = pv * 10
            pv = pv + 16
            return jnp.where(pv >= P, pv - P, pv)
        lax.fori_loop(0, XPC // 16, patg, lane)

        @pl.when(sid == 0)
        def _build_table():
            pltpu.sync_copy(digit_hbm, digit_v.at[pl.ds(0, 160)])
            pltpu.sync_copy(row_hbm, row_v.at[pl.ds(0, 72)])
            pltpu.sync_copy(col_hbm, col_v.at[pl.ds(0, 72)])
            for c in range(9):
                cv = col_v[pl.ds(c * 8, 16)]
                acc = cv * 0.0
                for i in range(8):
                    acc = jnp.where(lane == 8 + i, cv[i], acc)
                colhi_v[pl.ds(c * 16, 16)] = acc
            for p in range(P):
                r, c = p // 9, p % 9
                a = row_v[pl.ds(r * 8, 16)]
                b = colhi_v[pl.ds(c * 16, 16)]
                pos_v[pl.ds(p * 16, 16)] = jnp.where(lane < 8, a, b)
            def tb(p, carry):
                pv = pos_v[pl.ds(p * 16, 16)]
                for d in range(10):
                    tbl_v[p * 10 + d, pl.ds(0, 16)] = \
                        digit_v[pl.ds(d * 16, 16)]
                    tbl_v[p * 10 + d, pl.ds(16, 16)] = pv
                return carry
            lax.fori_loop(0, P, tb, 0)
            pltpu.sync_copy(tbl_v, stbl_v)
        plsc.subcore_barrier()

        def do_chunk(k, x_v, xsem, osem, chunk):
            @pl.when(chunk >= 2)
            def _drain_out():
                pltpu.make_async_copy(rows_v.at[k], out_slice(chunk), osem
                                      ).wait()
            pltpu.make_async_copy(x_slice(chunk), x_v.at[pl.ds(0, XPC)], xsem
                                  ).wait()

            @plsc.parallel_loop(0, XPC // 16, unroll=4)
            def cgrp(g):
                x_v[pl.ds(g * 16, 16)] = (
                    x_v[pl.ds(g * 16, 16)] + pat_v[pl.ds(g * 16, 16)])

            pltpu.async_copy(stbl_v.at[x_v], rows_v.at[k], gsem).wait()

            pltpu.async_copy(rows_v.at[k], out_slice(chunk), osem)

            @pl.when(chunk + 2 < n_chunks)
            def _prefetch_x():
                pltpu.async_copy(x_slice(chunk + 2), x_v.at[pl.ds(0, XPC)],
                                 xsem)

        def chunk_pair(ci, carry):
            do_chunk(0, x0_v, xsem0, osem0, ci * 2)
            do_chunk(1, x1_v, xsem1, osem1, ci * 2 + 1)
            return carry
        lax.fori_loop(0, n_chunks // 2, chunk_pair, 0)

        pltpu.make_async_copy(rows_v.at[0], out_slice(n_chunks - 2), osem0
                              ).wait()
        pltpu.make_async_copy(rows_v.at[1], out_slice(n_chunks - 1), osem1
                              ).wait()

    return enc


def kernel(x, digit_emb, row_emb, col_emb):
    B, p = x.shape
    assert p == P
    xf = x.reshape(-1).astype(jnp.int32)
    out = _make_encoder(B)(xf, digit_emb.reshape(-1),
                           row_emb.reshape(-1), col_emb.reshape(-1))
    return out.reshape(B, P, F)


# speedup vs baseline: 6.1965x; 5.8571x over previous
import functools

import jax
import jax.numpy as jnp
from jax import lax
from jax.experimental import pallas as pl
from jax.experimental.pallas import tpu as pltpu
from jax.experimental.pallas import tpu_sc as plsc

DIGIT_DIM = 16
POS_DIM = 8
P = 81
F = 32
NW = 32
NB = 16
ROW_W = P * F


@functools.lru_cache(maxsize=None)
def _make_encoder(B: int):
    assert B % (NW * NB) == 0
    rows_per_w = B // NW
    n_chunks = rows_per_w // NB
    x_per_chunk = NB * P
    out_per_chunk = NB * ROW_W
    x_per_w = rows_per_w * P

    mesh = plsc.VectorSubcoreMesh(core_axis_name="c", subcore_axis_name="s")

    @functools.partial(
        pl.kernel,
        mesh=mesh,
        out_type=jax.ShapeDtypeStruct((B * ROW_W,), jnp.float32),
        scratch_types=[
            pltpu.VMEM((160,), jnp.float32),
            pltpu.VMEM((80,), jnp.float32),
            pltpu.VMEM((80,), jnp.float32),
            pltpu.VMEM((9 * 16,), jnp.float32),
            pltpu.VMEM((P * 16,), jnp.float32),
            pltpu.VMEM((x_per_chunk,), jnp.int32),
            pltpu.VMEM((x_per_chunk,), jnp.int32),
            pltpu.VMEM((2, out_per_chunk), jnp.float32),
            pltpu.SemaphoreType.DMA,
            pltpu.SemaphoreType.DMA,
            pltpu.SemaphoreType.DMA,
            pltpu.SemaphoreType.DMA,
        ],
    )
    def enc(x_hbm, digit_hbm, row_hbm, col_hbm, out_hbm,
            digit_v, row_v, col_v, colhi_v, pos_v, x0_v, x1_v, out_v,
            xsem0, xsem1, osem0, osem1):
        wid = lax.axis_index("s") * 2 + lax.axis_index("c")
        lane = lax.iota(jnp.int32, 16)

        def x_slice(chunk):
            return x_hbm.at[pl.ds(wid * x_per_w + chunk * x_per_chunk,
                                  x_per_chunk)]

        def out_slice(chunk):
            return out_hbm.at[pl.ds((wid * x_per_w + chunk * x_per_chunk) * F,
                                    out_per_chunk)]

        pltpu.async_copy(x_slice(0), x0_v.at[pl.ds(0, x_per_chunk)], xsem0)
        pltpu.async_copy(x_slice(1), x1_v.at[pl.ds(0, x_per_chunk)], xsem1)

        pltpu.sync_copy(digit_hbm, digit_v.at[pl.ds(0, 160)])
        pltpu.sync_copy(row_hbm, row_v.at[pl.ds(0, 72)])
        pltpu.sync_copy(col_hbm, col_v.at[pl.ds(0, 72)])

        for c in range(9):
            cv = col_v[pl.ds(c * 8, 16)]
            acc = cv * 0.0
            for i in range(8):
                acc = jnp.where(lane == 8 + i, cv[i], acc)
            colhi_v[pl.ds(c * 16, 16)] = acc

        for p in range(P):
            r, c = p // 9, p % 9
            a = row_v[pl.ds(r * 8, 16)]
            b = colhi_v[pl.ds(c * 16, 16)]
            pos_v[pl.ds(p * 16, 16)] = jnp.where(lane < 8, a, b)

        @plsc.parallel_loop(0, NB)
        def fill_row(b):
            for p in range(P):
                v = pos_v[pl.ds(p * 16, 16)]
                for k in (0, 1):
                    out_v[k, pl.ds(b * ROW_W + p * F + DIGIT_DIM, 16)] = v

        def do_chunk(k, xk_v, xsem, osem, chunk):
            @pl.when(chunk >= 2)
            def _drain_out():
                pltpu.make_async_copy(out_v.at[k], out_slice(chunk), osem
                                      ).wait()
            pltpu.make_async_copy(x_slice(chunk),
                                  xk_v.at[pl.ds(0, x_per_chunk)], xsem).wait()

            @plsc.parallel_loop(0, P, unroll=2)
            def grp(g):
                xv = xk_v[pl.ds(g * 16, 16)] * DIGIT_DIM
                base_o = g * 512
                for j in range(16):
                    dv = digit_v[pl.ds(xv[j], DIGIT_DIM)]
                    out_v[k, pl.ds(base_o + j * F, 16)] = dv

            pltpu.async_copy(out_v.at[k], out_slice(chunk), osem)

            @pl.when(chunk + 2 < n_chunks)
            def _prefetch_x():
                pltpu.async_copy(x_slice(chunk + 2),
                                 xk_v.at[pl.ds(0, x_per_chunk)], xsem)

        def chunk_pair(ci, carry):
            do_chunk(0, x0_v, xsem0, osem0, ci * 2)
            do_chunk(1, x1_v, xsem1, osem1, ci * 2 + 1)
            return carry
        lax.fori_loop(0, n_chunks // 2, chunk_pair, 0)

        pltpu.make_async_copy(out_v.at[0], out_slice(n_chunks - 2), osem0
                              ).wait()
        pltpu.make_async_copy(out_v.at[1], out_slice(n_chunks - 1), osem1
                              ).wait()

    return enc


def kernel(x, digit_emb, row_emb, col_emb):
    B, p = x.shape
    assert p == P
    xf = x.reshape(-1).astype(jnp.int32)
    out = _make_encoder(B)(xf, digit_emb.reshape(-1),
                           row_emb.reshape(-1), col_emb.reshape(-1))
    return out.reshape(B, P, F)


# speedup vs baseline: 20.1227x; 3.2474x over previous
import functools

import jax
import jax.numpy as jnp
from jax import lax
from jax.experimental import pallas as pl
from jax.experimental.pallas import tpu as pltpu
from jax.experimental.pallas import tpu_sc as plsc

DIGIT_DIM = 16
POS_DIM = 8
P = 81
F = 32
NW = 32
BB = 128
PP = 8


@functools.lru_cache(maxsize=None)
def _make_encoder(B: int):
    bw = B // NW
    nb = bw // BB
    np0 = P // PP
    assert bw % BB == 0 and nb == 4

    mesh = plsc.VectorSubcoreMesh(core_axis_name="c", subcore_axis_name="s")

    @functools.partial(
        pl.kernel,
        mesh=mesh,
        out_type=jax.ShapeDtypeStruct((P, F, B), jnp.float32),
        scratch_types=[
            pltpu.VMEM((160,), jnp.float32),
            pltpu.VMEM((80,), jnp.float32),
            pltpu.VMEM((80,), jnp.float32),
            pltpu.VMEM((9 * 16,), jnp.float32),
            pltpu.VMEM((P * 16,), jnp.float32),
            pltpu.VMEM((10 * 16 * 16,), jnp.float32),
            pltpu.VMEM((PP * 16 * 16,), jnp.float32),
            pltpu.VMEM((PP, BB), jnp.int32),
            pltpu.VMEM((PP, BB), jnp.int32),
            pltpu.VMEM((PP, F, BB), jnp.float32),
            pltpu.VMEM((PP, F, BB), jnp.float32),
            pltpu.SemaphoreType.DMA,
            pltpu.SemaphoreType.DMA,
        ],
    )
    def enc(xt_hbm, digit_hbm, row_hbm, col_hbm, out_hbm,
            digit_v, row_v, col_v, colhi_v, pos_v, sd_v, tpos_v,
            x0_v, x1_v, st0_v, st1_v, osem0, osem1):
        wid = lax.axis_index("s") * 2 + lax.axis_index("c")
        lane = lax.iota(jnp.int32, 16)
        zerof = lane * jnp.float32(0)
        b_base = wid * bw
        bufs = ((x0_v, st0_v, osem0), (x1_v, st1_v, osem1))

        pltpu.sync_copy(digit_hbm, digit_v.at[pl.ds(0, 160)])
        pltpu.sync_copy(row_hbm, row_v.at[pl.ds(0, 72)])
        pltpu.sync_copy(col_hbm, col_v.at[pl.ds(0, 72)])

        for d in range(10):
            dv = digit_v[pl.ds(d * 16, 16)]
            for f in range(DIGIT_DIM):
                sd_v[pl.ds((d * 16 + f) * 16, 16)] = zerof + dv[f]

        for c in range(9):
            cv = col_v[pl.ds(c * 8, 16)]
            acc = cv * 0.0
            for i in range(8):
                acc = jnp.where(lane == 8 + i, cv[i], acc)
            colhi_v[pl.ds(c * 16, 16)] = acc

        for p in range(P):
            r, c = p // 9, p % 9
            a = row_v[pl.ds(r * 8, 16)]
            b = colhi_v[pl.ds(c * 16, 16)]
            pos_v[pl.ds(p * 16, 16)] = jnp.where(lane < 8, a, b)

        def build_tpos(p0, ppc):
            def one(pl_, carry):
                pv = pos_v[pl.ds((p0 + pl_) * 16, 16)]
                for f in range(16):
                    tpos_v[pl.ds(pl_ * 256 + f * 16, 16)] = zerof + pv[f]
                return carry
            lax.fori_loop(0, ppc, one, 0)

        def fill_template(st_v, ppc):
            def one(pl_, carry):
                for f in range(16):
                    v = tpos_v[pl.ds(pl_ * 256 + f * 16, 16)]
                    for g in range(BB // 16):
                        st_v[pl_, DIGIT_DIM + f, pl.ds(g * 16, 16)] = v
                return carry
            lax.fori_loop(0, ppc, one, 0)

        def compute_block(x_v, st_v, ppc):
            for fb in range(4):
                sd = [[sd_v[pl.ds((d * 16 + fb * 4 + fi) * 16, 16)]
                       for d in range(10)] for fi in range(4)]

                def one(pl_, carry):
                    def grp(g, c2):
                        xv = x_v[pl_, pl.ds(g * 16, 16)]
                        ms = [xv == d for d in range(1, 10)]
                        for fi in range(4):
                            acc = sd[fi][0]
                            for d in range(1, 10):
                                acc = jnp.where(ms[d - 1], sd[fi][d], acc)
                            st_v[pl_, fb * 4 + fi, pl.ds(g * 16, 16)] = acc
                        return c2
                    lax.fori_loop(0, BB // 16, grp, 0)
                    return carry
                lax.fori_loop(0, ppc, one, 0)

        def do_batch_block(p0, k, bi, drain_first):
            x_v, st_v, osem = bufs[k]
            oslice = out_hbm.at[pl.ds(p0, PP), :,
                                pl.ds(b_base + bi * BB, BB)]
            if drain_first:
                pltpu.make_async_copy(st_v, oslice, osem).wait()
            pltpu.sync_copy(
                xt_hbm.at[pl.ds(p0, PP), pl.ds(b_base + bi * BB, BB)], x_v)
            compute_block(x_v, st_v, PP)
            pltpu.async_copy(st_v, oslice, osem)

        def main_pc(pc, carry):
            p0 = pc * PP
            build_tpos(p0, PP)
            for k in (0, 1):
                x_v, st_v, osem = bufs[k]

                @pl.when(pc >= 1)
                def _drain():
                    pltpu.make_async_copy(
                        st_v,
                        out_hbm.at[pl.ds(p0, PP), :,
                                   pl.ds(b_base + k * BB, BB)],
                        osem).wait()
                fill_template(st_v, PP)
            for bi in range(nb):
                do_batch_block(p0, bi % 2, bi, drain_first=(bi >= 2))
            return carry
        lax.fori_loop(0, np0, main_pc, 0)

        for k in (0, 1):
            x_v, st_v, osem = bufs[k]
            pltpu.make_async_copy(
                st_v, out_hbm.at[pl.ds(0, PP), :, pl.ds(b_base, BB)],
                osem).wait()

        p0t = P - 1
        build_tpos(p0t, 1)
        for k in (0, 1):
            fill_template(bufs[k][1], 1)
        for bi in range(nb):
            x_v, st_v, osem = bufs[bi % 2]
            st_t = st_v.at[pl.ds(0, 1)]
            oslice = out_hbm.at[pl.ds(p0t, 1), :,
                                pl.ds(b_base + bi * BB, BB)]
            if bi >= 2:
                pltpu.make_async_copy(st_t, oslice, osem).wait()
            pltpu.sync_copy(
                xt_hbm.at[pl.ds(p0t, 1), pl.ds(b_base + bi * BB, BB)],
                x_v.at[pl.ds(0, 1)])
            compute_block(x_v, st_v, 1)
            pltpu.async_copy(st_t, oslice, osem)
        for k in (0, 1):
            x_v, st_v, osem = bufs[k]
            pltpu.make_async_copy(
                st_v.at[pl.ds(0, 1)],
                out_hbm.at[pl.ds(p0t, 1), :, pl.ds(b_base, BB)], osem).wait()

    return enc


def kernel(x, digit_emb, row_emb, col_emb):
    B, p = x.shape
    assert p == P
    xt = x.astype(jnp.int32).T
    out_t = _make_encoder(B)(xt, digit_emb.reshape(-1),
                             row_emb.reshape(-1), col_emb.reshape(-1))
    return jnp.transpose(out_t, (2, 0, 1))
